# baseline (speedup 1.0000x reference)
import functools

import jax
import jax.numpy as jnp
from jax import lax
from jax.experimental import pallas as pl
from jax.experimental.pallas import tpu as pltpu
from jax.experimental.pallas import tpu_sc as plsc

N = 10000
E = 160000
F_IN = 256
F_OUT = 256
B = 4
K = 4
W = B * F_OUT
NQ = 8
KN = K * N

NS = 16
L = 16

CH = 512
NCHUNK = 20
NPAD = CH * NCHUNK
ET = E // NS
FB = 2000
GBLK = 2048
PKW = 10240
MAILW = 12064
MB = 64

NB = 1000


def _mm_body(x_ref, w_ref, o_ref):
    o_ref[0] = jnp.dot(x_ref[0], w_ref[0, 0],
                       preferred_element_type=jnp.float32)


def _dense_phase(x, w2):
    return pl.pallas_call(
        _mm_body,
        grid=(K, N // NB, NQ),
        in_specs=[
            pl.BlockSpec((1, NB, F_IN), lambda i, n, q: (q // 2, n, 0)),
            pl.BlockSpec((1, 1, F_IN, 128), lambda i, n, q: (i, q % 2, 0, 0)),
        ],
        out_specs=pl.BlockSpec(
            (1, NB, 128), lambda i, n, q: (q, i * (N // NB) + n, 0)),
        out_shape=jax.ShapeDtypeStruct((NQ, KN, 128), jnp.float32),
    )(x, w2)


def _spmm_body(res_hbm, dst_hbm, src_hbm, val_hbm, out_hbm,
               pkb, valb, es, ed, ev, rows, acc,
               mpk, mval, accx, counts):
    cid = lax.axis_index("c")
    sid = lax.axis_index("s")
    e8 = (sid // 8) * 8
    q = sid % 8
    qbase = q * KN
    iot = jnp.arange(L, dtype=jnp.int32)
    zf = jnp.zeros((L,), jnp.float32)
    zi = jnp.zeros((L,), jnp.int32)
    ones_m = iot >= 0

    def support_pass(i, lo):
        for m in range(NS):
            counts[m] = 0
        plsc.subcore_barrier()

        ibase = i * N

        def fblk(blk, cnt):
            o = pl.multiple_of(blk * FB, 16)
            ebase = pl.multiple_of(i * E + sid * ET + o, 8)
            pltpu.sync_copy(dst_hbm.at[pl.ds(ebase, FB)], ed.at[pl.ds(0, FB)])
            pltpu.sync_copy(src_hbm.at[pl.ds(ebase, FB)], es.at[pl.ds(0, FB)])
            pltpu.sync_copy(val_hbm.at[pl.ds(ebase, FB)], ev.at[pl.ds(0, FB)])

            def fgroup(j, cnt):
                jb = pl.multiple_of(j * L, L)
                d = ed[pl.ds(jb, L)]
                s = es[pl.ds(jb, L)]
                v = ev[pl.ds(jb, L)]
                m = (d >= lo) & (d < lo + CH)
                mi = m.astype(jnp.int32)
                pos = cnt + plsc.cumsum(mi) - mi
                pk = (s + ibase) * 1024 + (d - lo)
                plsc.store_scatter(pkb, (pos,), pk, mask=m)
                plsc.store_scatter(valb, (pos,), v, mask=m)
                return cnt + jnp.sum(mi)

            return lax.fori_loop(0, FB // L, fgroup, cnt)

        cnt = lax.fori_loop(0, ET // FB, fblk, jnp.int32(0))

        p = cnt + iot
        plsc.store_scatter(pkb, (p,), zi, mask=ones_m)
        plsc.store_scatter(valb, (p,), zf, mask=ones_m)
        cntp = ((cnt + L - 1) // L) * L

        for nb_t in range(NS):
            plsc.fetch_and_add(counts.at[sid], cntp, subcore_id=nb_t)
        mbase = sid * MAILW

        def pub(blk, _):
            o = pl.multiple_of(blk * GBLK, 16)
            mo = pl.multiple_of(mbase + o, 16)
            pltpu.sync_copy(pkb.at[pl.ds(o, GBLK)], mpk.at[pl.ds(mo, GBLK)])
            pltpu.sync_copy(valb.at[pl.ds(o, GBLK)], mval.at[pl.ds(mo, GBLK)])
            return _

        lax.fori_loop(0, (cntp + GBLK - 1) // GBLK, pub, jnp.int32(0))
        plsc.subcore_barrier()

        def consume_mail(mm, _):
            m = e8 + mm
            cm = counts[m]
            mb = m * MAILW

            def blk_body(blk, _):
                o = pl.multiple_of(blk * GBLK, 16)
                mo = pl.multiple_of(mb + o, 16)
                pltpu.sync_copy(mpk.at[pl.ds(mo, GBLK)], es.at[pl.ds(0, GBLK)])
                pltpu.sync_copy(mval.at[pl.ds(mo, GBLK)], ev.at[pl.ds(0, GBLK)])
                ngrp = jnp.minimum(cm - o, GBLK) // L

                def grp(g, _):
                    gb = pl.multiple_of(g * L, L)
                    pk16 = es[pl.ds(gb, L)]
                    v16 = ev[pl.ds(gb, L)]
                    s16 = lax.shift_right_logical(pk16, 10)
                    d16 = pk16 & 1023
                    pltpu.sync_copy(res_hbm.at[s16 + qbase], rows)
                    for ln in range(L):
                        sp = jnp.full((L,), ln, jnp.int32)
                        db = d16.at[sp].get(mode='promise_in_bounds')
                        vb = v16.at[sp].get(mode='promise_in_bounds')
                        for f in range(8):
                            sl = rows[ln, pl.ds(f * L, L)] * vb
                            plsc.addupdate_scatter(acc, (db, iot + f * L), sl)
                    return _

                lax.fori_loop(0, ngrp, grp, jnp.int32(0))
                return _

            lax.fori_loop(0, (cm + GBLK - 1) // GBLK, blk_body, jnp.int32(0))
            return _

        lax.fori_loop(0, 8, consume_mail, jnp.int32(0))
        plsc.subcore_barrier()
        return lo

    def chunk_body(kc, _):
        lo = (kc * 2 + cid) * CH

        def zrow(r, _):
            for f in range(8):
                acc[r, pl.ds(f * L, L)] = zf
            return _

        lax.fori_loop(0, CH, zrow, jnp.int32(0))

        lax.fori_loop(0, K, support_pass, lo)

        def merge_sb(sb, _):
            so = pl.multiple_of(sb * MB, 16)

            @pl.when(sid >= 8)
            def _ship():
                pltpu.sync_copy(acc.at[pl.ds(so, MB)], accx.at[sid - 8])

            plsc.subcore_barrier()

            @pl.when(sid < 8)
            def _merge():
                def mrow(rb, _):
                    rbo = pl.multiple_of(rb * L, L)
                    pltpu.sync_copy(accx.at[sid, pl.ds(rbo, L)], rows)
                    for r in range(L):
                        for f in range(8):
                            sl = pl.ds(f * L, L)
                            a = acc[so + rbo + r, sl] + rows[r, sl]
                            acc[so + rbo + r, sl] = a
                    return _

                lax.fori_loop(0, MB // L, mrow, jnp.int32(0))

            plsc.subcore_barrier()
            return _

        lax.fori_loop(0, CH // MB, merge_sb, jnp.int32(0))

        @pl.when(sid < 8)
        def _writeout():
            pltpu.sync_copy(acc, out_hbm.at[pl.ds(q * NPAD + lo, CH)])

        return _

    lax.fori_loop(0, NCHUNK // 2, chunk_body, jnp.int32(0))


def _sparse_phase(res8, dst_all, src_all, val_all):
    mesh = plsc.VectorSubcoreMesh(core_axis_name="c", subcore_axis_name="s")
    return pl.kernel(
        _spmm_body,
        out_type=jax.ShapeDtypeStruct((NQ * NPAD, 128), jnp.float32),
        mesh=mesh,
        compiler_params=pltpu.CompilerParams(needs_layout_passes=False),
        scratch_types=[
            pltpu.VMEM((PKW,), jnp.int32),
            pltpu.VMEM((PKW,), jnp.float32),
            pltpu.VMEM((GBLK,), jnp.int32),
            pltpu.VMEM((GBLK,), jnp.int32),
            pltpu.VMEM((GBLK,), jnp.float32),
            pltpu.VMEM((L, 128), jnp.float32),
            pltpu.VMEM((CH, 128), jnp.float32),
            pltpu.VMEM_SHARED((NS * MAILW,), jnp.int32),
            pltpu.VMEM_SHARED((NS * MAILW,), jnp.float32),
            pltpu.VMEM_SHARED((8, MB, 128), jnp.float32),
            pltpu.SMEM((NS,), jnp.int32),
        ],
    )(res8, dst_all, src_all, val_all)


def kernel(x, sup0_idx, sup0_val, kernel0, sup1_idx, sup1_val, kernel1,
           sup2_idx, sup2_val, kernel2, sup3_idx, sup3_val, kernel3, bias):
    w_all = jnp.stack([kernel0, kernel1, kernel2, kernel3])
    w2 = w_all.reshape(K, F_IN, 2, 128).transpose(0, 2, 1, 3)
    res = _dense_phase(x, w2)
    res8 = res.reshape(NQ * KN, 128)

    idxs = (sup0_idx, sup1_idx, sup2_idx, sup3_idx)
    dst_all = jnp.concatenate([s[0] for s in idxs])
    src_all = jnp.concatenate([s[1] for s in idxs])
    val_all = jnp.concatenate([sup0_val, sup1_val, sup2_val, sup3_val])

    out_t = _sparse_phase(res8, dst_all, src_all, val_all)
    out = out_t.reshape(NQ, NPAD, 128)[:, :N].transpose(1, 0, 2)
    out = out.reshape(N, B, F_OUT).transpose(1, 0, 2) + bias
    return out
